# baseline (speedup 1.0000x reference)
import functools

import jax
import jax.numpy as jnp
from jax import lax
from jax.experimental import pallas as pl
from jax.experimental.pallas import tpu as pltpu
from jax.experimental.pallas import tpu_sc as plsc

NC = 2
NS = 16
L = 16
NW = NC * NS

D = 128
CH = 64
DCH = 80
PBITS = 14

@functools.cache
def _mesh():
    return plsc.VectorSubcoreMesh(
        core_axis_name="c", subcore_axis_name="s", num_cores=NC, num_subcores=NS)


def _make_deg(np_, nch):
    rows_t = np_ // NS

    @functools.partial(
        pl.kernel,
        mesh=_mesh(),
        compiler_params=pltpu.CompilerParams(needs_layout_passes=False),
        out_type=jax.ShapeDtypeStruct((NC, np_, D), jnp.float32),
        scratch_types=[
            pltpu.VMEM_SHARED((np_,), jnp.float32),
            pltpu.VMEM((nch, CH), jnp.int32),
            pltpu.VMEM((-(-CH // L) * L,), jnp.float32),
            pltpu.VMEM((rows_t,), jnp.float32),
            pltpu.VMEM((rows_t, D), jnp.float32),
        ],
    )
    def deg_kernel(ei_hbm, deg_hbm, hist_sp, didx_v, ones_v, dvm_v, dbc_v):
        c = lax.axis_index("c")
        s = lax.axis_index("s")
        w = c * NS + s

        for k in range(rows_t // L):
            dvm_v[pl.ds(k * L, L)] = jnp.zeros((L,), jnp.float32)
        for k in range(-(-CH // L)):
            ones_v[pl.ds(k * L, L)] = jnp.ones((L,), jnp.float32)
        pltpu.sync_copy(dvm_v, hist_sp.at[pl.ds(s * rows_t, rows_t)])
        pltpu.sync_copy(ei_hbm.at[1, w], didx_v)
        plsc.subcore_barrier()

        def win(j, carry):
            pltpu.sync_copy(ones_v.at[pl.ds(0, CH)],
                            hist_sp.at[didx_v.at[j]], add=True)
            return carry
        lax.fori_loop(0, nch, win, 0)
        plsc.subcore_barrier()

        pltpu.sync_copy(hist_sp.at[pl.ds(s * rows_t, rows_t)], dvm_v)

        def rowgrp(g, carry):
            vec = dvm_v[pl.ds(g * L, L)]
            for i in range(L):
                bc = jnp.full((L,), vec[i], jnp.float32)
                for k in range(D // L):
                    dbc_v[g * L + i, pl.ds(k * L, L)] = bc
            return carry
        lax.fori_loop(0, rows_t // L, rowgrp, 0)
        pltpu.sync_copy(dbc_v, deg_hbm.at[c, pl.ds(s * rows_t, rows_t), :])

    return deg_kernel


NB = 2

NR = 2 * NB

def _make_agg(np_, nch):
    rows_t = np_ // NS
    assert nch % NB == 0

    @functools.partial(
        pl.kernel,
        mesh=_mesh(),
        compiler_params=pltpu.CompilerParams(needs_layout_passes=False),
        out_type=jax.ShapeDtypeStruct((NC, np_, D), jnp.float32),
        scratch_types=[
            pltpu.VMEM_SHARED((np_, D), jnp.float32),
            pltpu.VMEM((nch, CH), jnp.int32),
            pltpu.VMEM((NR, CH), jnp.int32),
            pltpu.VMEM((NR, CH), jnp.int32),
            pltpu.VMEM((NB, CH, D), jnp.float32),
            [pltpu.SemaphoreType.DMA] * NB,
            [pltpu.SemaphoreType.DMA] * NB,
        ],
    )
    def agg_kernel(hp_hbm, pe_hbm, acc_hbm, acc_sp, pk_v, sidx_v, didx_v,
                   rows_v, gsems, ssems):
        c = lax.axis_index("c")
        s = lax.axis_index("s")
        w = c * NS + s

        pltpu.sync_copy(hp_hbm.at[pl.ds(s * rows_t, rows_t), :],
                        acc_sp.at[pl.ds(s * rows_t, rows_t), :])
        pltpu.sync_copy(pe_hbm.at[w], pk_v)
        plsc.subcore_barrier()

        def unpack(j, slot):
            for k in range(CH // L):
                v = pk_v[j, pl.ds(k * L, L)]
                sidx_v[slot, pl.ds(k * L, L)] = lax.shift_right_logical(
                    v, PBITS)
                didx_v[slot, pl.ds(k * L, L)] = lax.bitwise_and(
                    v, (1 << PBITS) - 1)

        for b in range(NB):
            unpack(b, b)
            pltpu.async_copy(hp_hbm.at[sidx_v.at[b]], rows_v.at[b], gsems[b])

        def grp(g, carry):
            for b in range(NB):
                j = g * NB + b
                pltpu.make_async_copy(
                    hp_hbm.at[sidx_v.at[j % NR]], rows_v.at[b],
                    gsems[b]).wait()
                pltpu.async_copy(
                    rows_v.at[b], acc_sp.at[didx_v.at[j % NR]], ssems[b],
                    add=True)
            for b in range(NB):
                j = g * NB + b + NB

                @pl.when(j < nch)
                def _(b=b, j=j):
                    pltpu.make_async_copy(
                        rows_v.at[b], acc_sp.at[didx_v.at[(j - NB) % NR]],
                        ssems[b]).wait()
                    unpack(j, j % NR)
                    pltpu.async_copy(
                        hp_hbm.at[sidx_v.at[j % NR]], rows_v.at[b], gsems[b])
            return carry
        lax.fori_loop(0, nch // NB, grp, 0)
        for b in range(NB):
            pltpu.make_async_copy(
                rows_v.at[b], acc_sp.at[didx_v.at[(nch - NB + b) % NR]],
                ssems[b]).wait()
        plsc.subcore_barrier()

        pltpu.sync_copy(acc_sp.at[pl.ds(s * rows_t, rows_t), :],
                        acc_hbm.at[c, pl.ds(s * rows_t, rows_t), :])

    return agg_kernel


def _make_decode(np_, el):
    nt = el // DCH
    jmax = -(-nt // NW)

    @functools.partial(
        pl.kernel,
        mesh=_mesh(),
        compiler_params=pltpu.CompilerParams(needs_layout_passes=False),
        out_type=jax.ShapeDtypeStruct((nt, 1, DCH), jnp.float32),
        scratch_types=[
            pltpu.VMEM((1, DCH), jnp.int32),
            pltpu.VMEM((1, DCH), jnp.int32),
            pltpu.VMEM((DCH, D), jnp.float32),
            pltpu.VMEM((DCH, D), jnp.float32),
            pltpu.VMEM((1, DCH), jnp.float32),
            pltpu.SemaphoreType.DMA,
        ],
    )
    def dec_kernel(z_hbm, eli_hbm, out_hbm, aidx, bidx, za, zb, outv, sem):
        c = lax.axis_index("c")
        s = lax.axis_index("s")
        w = c * NS + s

        def window(j, carry):
            t = w + NW * j

            @pl.when(t < nt)
            def _():
                pltpu.sync_copy(eli_hbm.at[0, t], aidx)
                pltpu.sync_copy(eli_hbm.at[1, t], bidx)
                pltpu.async_copy(z_hbm.at[aidx.at[0]], za, sem).wait()
                pltpu.async_copy(z_hbm.at[bidx.at[0]], zb, sem).wait()

                lanes = lax.iota(jnp.int32, L)

                def group(g, carry2):
                    res = jnp.zeros((L,), jnp.float32)
                    for i in range(L):
                        p = g * L + i
                        acc = jnp.zeros((L,), jnp.float32)
                        for k in range(D // L):
                            acc = acc + (za[p, pl.ds(k * L, L)]
                                         * zb[p, pl.ds(k * L, L)])
                        tot = jnp.sum(acc)
                        res = jnp.where(lanes == i, tot, res)
                    outv[0, pl.ds(g * L, L)] = res
                    return carry2
                lax.fori_loop(0, DCH // L, group, 0)
                pltpu.sync_copy(outv, out_hbm.at[t])
            return carry
        lax.fori_loop(0, jmax, window, 0)

    return dec_kernel


def _tc_block(np_):
    blk = 1280
    grid = np_ // blk
    return blk, grid


def _make_tc1(np_):
    blk, grid = _tc_block(np_)

    def body(x_ref, w_ref, deg_ref, h_ref, hp_ref):
        s = lax.rsqrt(deg_ref[0] + deg_ref[1] + 1.0)
        h = jnp.dot(x_ref[...], w_ref[...], preferred_element_type=jnp.float32)
        h_ref[...] = h
        hp_ref[...] = h * s

    return pl.pallas_call(
        body,
        grid=(grid,),
        in_specs=[
            pl.BlockSpec((blk, D), lambda i: (i, 0)),
            pl.BlockSpec((D, D), lambda i: (0, 0)),
            pl.BlockSpec((NC, blk, D), lambda i: (0, i, 0)),
        ],
        out_specs=[
            pl.BlockSpec((blk, D), lambda i: (i, 0)),
            pl.BlockSpec((blk, D), lambda i: (i, 0)),
        ],
        out_shape=[
            jax.ShapeDtypeStruct((np_, D), jnp.float32),
            jax.ShapeDtypeStruct((np_, D), jnp.float32),
        ],
    )


def _make_tc2(np_):
    blk, grid = _tc_block(np_)

    def body(acc_ref, h1_ref, deg_ref, w2_ref, b1_ref, h2_ref, hp2_ref):
        s = lax.rsqrt(deg_ref[0] + deg_ref[1] + 1.0)
        a = acc_ref[0] + acc_ref[1]
        h1 = h1_ref[...]
        act = jnp.maximum(s * a - (s * s) * h1 + b1_ref[...], 0.0)
        h2 = jnp.dot(act, w2_ref[...], preferred_element_type=jnp.float32)
        h2_ref[...] = h2
        hp2_ref[...] = h2 * s

    return pl.pallas_call(
        body,
        grid=(grid,),
        in_specs=[
            pl.BlockSpec((NC, blk, D), lambda i: (0, i, 0)),
            pl.BlockSpec((blk, D), lambda i: (i, 0)),
            pl.BlockSpec((NC, blk, D), lambda i: (0, i, 0)),
            pl.BlockSpec((D, D), lambda i: (0, 0)),
            pl.BlockSpec((1, D), lambda i: (0, 0)),
        ],
        out_specs=[
            pl.BlockSpec((blk, D), lambda i: (i, 0)),
            pl.BlockSpec((blk, D), lambda i: (i, 0)),
        ],
        out_shape=[
            jax.ShapeDtypeStruct((np_, D), jnp.float32),
            jax.ShapeDtypeStruct((np_, D), jnp.float32),
        ],
    )


def _make_tc3(np_):
    blk, grid = _tc_block(np_)

    def body(acc_ref, h2_ref, deg_ref, b2_ref, z_ref):
        s = lax.rsqrt(deg_ref[0] + deg_ref[1] + 1.0)
        a = acc_ref[0] + acc_ref[1]
        z_ref[...] = s * a - (s * s) * h2_ref[...] + b2_ref[...]

    return pl.pallas_call(
        body,
        grid=(grid,),
        in_specs=[
            pl.BlockSpec((NC, blk, D), lambda i: (0, i, 0)),
            pl.BlockSpec((blk, D), lambda i: (i, 0)),
            pl.BlockSpec((NC, blk, D), lambda i: (0, i, 0)),
            pl.BlockSpec((1, D), lambda i: (0, 0)),
        ],
        out_specs=pl.BlockSpec((blk, D), lambda i: (i, 0)),
        out_shape=jax.ShapeDtypeStruct((np_, D), jnp.float32),
    )


def kernel(x, edge_index, edge_label_index, W1, b1, W2, b2):
    n, d = x.shape
    e = edge_index.shape[1]
    el = edge_label_index.shape[1]
    assert d == D
    np_ = ((n + 1279) // 1280) * 1280

    nch = -(-e // (NW * CH))
    nch += (-nch) % NB
    e_pad = NW * CH * nch
    if e_pad > e and np_ == n:
        np_ += 1280
    ei32 = edge_index.astype(jnp.int32)
    dummy = n + (jnp.arange(e_pad - e, dtype=jnp.int32) % (np_ - n))
    eip = jnp.concatenate(
        [ei32, jnp.stack([dummy, dummy])], axis=1)
    ei = eip.reshape(2, NW, nch, CH)
    pe = ((eip[0] << PBITS) | eip[1]).reshape(NW, nch, CH)
    eli = edge_label_index.astype(jnp.int32).reshape(2, el // DCH, 1, DCH)
    xp = jnp.pad(x, ((0, np_ - n), (0, 0)))
    b1r = b1.reshape(1, D)
    b2r = b2.reshape(1, D)

    degs = _make_deg(np_, nch)(ei)
    h1, hp1 = _make_tc1(np_)(xp, W1, degs)
    acc1 = _make_agg(np_, nch)(hp1, pe)
    h2, hp2 = _make_tc2(np_)(acc1, h1, degs, W2, b1r)
    acc2 = _make_agg(np_, nch)(hp2, pe)
    z = _make_tc3(np_)(acc2, h2, degs, b2r)
    out = _make_decode(np_, el)(z, eli)
    return out.reshape(el)


# speedup vs baseline: 1.1035x; 1.1035x over previous
import functools

import jax
import jax.numpy as jnp
from jax import lax
from jax.experimental import pallas as pl
from jax.experimental.pallas import tpu as pltpu
from jax.experimental.pallas import tpu_sc as plsc

NC = 2
NS = 16
L = 16
NW = NC * NS

D = 128
CH = 128
DCH = 80
PBITS = 14

@functools.cache
def _mesh():
    return plsc.VectorSubcoreMesh(
        core_axis_name="c", subcore_axis_name="s", num_cores=NC, num_subcores=NS)


def _make_deg(np_, nch):
    rows_t = np_ // NS

    @functools.partial(
        pl.kernel,
        mesh=_mesh(),
        compiler_params=pltpu.CompilerParams(needs_layout_passes=False),
        out_type=jax.ShapeDtypeStruct((NC, np_, D), jnp.float32),
        scratch_types=[
            pltpu.VMEM_SHARED((np_,), jnp.float32),
            pltpu.VMEM((nch, CH), jnp.int32),
            pltpu.VMEM((-(-CH // L) * L,), jnp.float32),
            pltpu.VMEM((rows_t,), jnp.float32),
            pltpu.VMEM((rows_t, D), jnp.float32),
        ],
    )
    def deg_kernel(ei_hbm, deg_hbm, hist_sp, didx_v, ones_v, dvm_v, dbc_v):
        c = lax.axis_index("c")
        s = lax.axis_index("s")
        w = c * NS + s

        for k in range(rows_t // L):
            dvm_v[pl.ds(k * L, L)] = jnp.zeros((L,), jnp.float32)
        for k in range(-(-CH // L)):
            ones_v[pl.ds(k * L, L)] = jnp.ones((L,), jnp.float32)
        pltpu.sync_copy(dvm_v, hist_sp.at[pl.ds(s * rows_t, rows_t)])
        pltpu.sync_copy(ei_hbm.at[1, w], didx_v)
        plsc.subcore_barrier()

        def win(j, carry):
            pltpu.sync_copy(ones_v.at[pl.ds(0, CH)],
                            hist_sp.at[didx_v.at[j]], add=True)
            return carry
        lax.fori_loop(0, nch, win, 0)
        plsc.subcore_barrier()

        pltpu.sync_copy(hist_sp.at[pl.ds(s * rows_t, rows_t)], dvm_v)

        def rowgrp(g, carry):
            vec = dvm_v[pl.ds(g * L, L)]
            for i in range(L):
                bc = jnp.full((L,), vec[i], jnp.float32)
                for k in range(D // L):
                    dbc_v[g * L + i, pl.ds(k * L, L)] = bc
            return carry
        lax.fori_loop(0, rows_t // L, rowgrp, 0)
        pltpu.sync_copy(dbc_v, deg_hbm.at[c, pl.ds(s * rows_t, rows_t), :])

    return deg_kernel


NB = 2

NR = 2 * NB

def _make_agg(np_, nch):
    rows_t = np_ // NS
    assert nch % NB == 0

    @functools.partial(
        pl.kernel,
        mesh=_mesh(),
        compiler_params=pltpu.CompilerParams(needs_layout_passes=False),
        out_type=jax.ShapeDtypeStruct((NC, np_, D), jnp.float32),
        scratch_types=[
            pltpu.VMEM_SHARED((np_, D), jnp.float32),
            pltpu.VMEM((nch, CH), jnp.int32),
            pltpu.VMEM((NR, CH), jnp.int32),
            pltpu.VMEM((NR, CH), jnp.int32),
            pltpu.VMEM((NB, CH, D), jnp.float32),
            [pltpu.SemaphoreType.DMA] * NB,
            [pltpu.SemaphoreType.DMA] * NB,
            pltpu.SemaphoreType.DMA,
        ],
    )
    def agg_kernel(hp_hbm, pe_hbm, acc_hbm, acc_sp, pk_v, sidx_v, didx_v,
                   rows_v, gsems, ssems, isem):
        c = lax.axis_index("c")
        s = lax.axis_index("s")
        w = c * NS + s

        pltpu.async_copy(hp_hbm.at[pl.ds(s * rows_t, rows_t), :],
                         acc_sp.at[pl.ds(s * rows_t, rows_t), :], isem)
        pltpu.sync_copy(pe_hbm.at[w], pk_v)
        pltpu.make_async_copy(
            hp_hbm.at[pl.ds(s * rows_t, rows_t), :],
            acc_sp.at[pl.ds(s * rows_t, rows_t), :], isem).wait()
        plsc.subcore_barrier()

        def unpack(j, slot):
            for k in range(CH // L):
                v = pk_v[j, pl.ds(k * L, L)]
                sidx_v[slot, pl.ds(k * L, L)] = lax.shift_right_logical(
                    v, PBITS)
                didx_v[slot, pl.ds(k * L, L)] = lax.bitwise_and(
                    v, (1 << PBITS) - 1)

        for b in range(NB):
            unpack(b, b)
            pltpu.async_copy(hp_hbm.at[sidx_v.at[b]], rows_v.at[b], gsems[b])

        def grp(g, carry):
            for b in range(NB):
                j = g * NB + b
                pltpu.make_async_copy(
                    hp_hbm.at[sidx_v.at[j % NR]], rows_v.at[b],
                    gsems[b]).wait()
                pltpu.async_copy(
                    rows_v.at[b], acc_sp.at[didx_v.at[j % NR]], ssems[b],
                    add=True)
            for b in range(NB):
                j = g * NB + b + NB

                @pl.when(j < nch)
                def _(b=b, j=j):
                    pltpu.make_async_copy(
                        rows_v.at[b], acc_sp.at[didx_v.at[(j - NB) % NR]],
                        ssems[b]).wait()
                    unpack(j, j % NR)
                    pltpu.async_copy(
                        hp_hbm.at[sidx_v.at[j % NR]], rows_v.at[b], gsems[b])
            return carry
        lax.fori_loop(0, nch // NB, grp, 0)
        for b in range(NB):
            pltpu.make_async_copy(
                rows_v.at[b], acc_sp.at[didx_v.at[(nch - NB + b) % NR]],
                ssems[b]).wait()
        plsc.subcore_barrier()

        pltpu.sync_copy(acc_sp.at[pl.ds(s * rows_t, rows_t), :],
                        acc_hbm.at[c, pl.ds(s * rows_t, rows_t), :])

    return agg_kernel


def _make_decode(np_, el):
    nt = el // DCH
    jmax = -(-nt // NW)

    @functools.partial(
        pl.kernel,
        mesh=_mesh(),
        compiler_params=pltpu.CompilerParams(needs_layout_passes=False),
        out_type=jax.ShapeDtypeStruct((nt, 1, DCH), jnp.float32),
        scratch_types=[
            pltpu.VMEM((1, DCH), jnp.int32),
            pltpu.VMEM((1, DCH), jnp.int32),
            pltpu.VMEM((DCH, D), jnp.float32),
            pltpu.VMEM((DCH, D), jnp.float32),
            pltpu.VMEM((1, DCH), jnp.float32),
            pltpu.SemaphoreType.DMA,
        ],
    )
    def dec_kernel(z_hbm, eli_hbm, out_hbm, aidx, bidx, za, zb, outv, sem):
        c = lax.axis_index("c")
        s = lax.axis_index("s")
        w = c * NS + s

        def window(j, carry):
            t = w + NW * j

            @pl.when(t < nt)
            def _():
                pltpu.sync_copy(eli_hbm.at[0, t], aidx)
                pltpu.sync_copy(eli_hbm.at[1, t], bidx)
                pltpu.async_copy(z_hbm.at[aidx.at[0]], za, sem).wait()
                pltpu.async_copy(z_hbm.at[bidx.at[0]], zb, sem).wait()

                lanes = lax.iota(jnp.int32, L)

                def group(g, carry2):
                    res = jnp.zeros((L,), jnp.float32)
                    for i in range(L):
                        p = g * L + i
                        acc = jnp.zeros((L,), jnp.float32)
                        for k in range(D // L):
                            acc = acc + (za[p, pl.ds(k * L, L)]
                                         * zb[p, pl.ds(k * L, L)])
                        tot = jnp.sum(acc)
                        res = jnp.where(lanes == i, tot, res)
                    outv[0, pl.ds(g * L, L)] = res
                    return carry2
                lax.fori_loop(0, DCH // L, group, 0)
                pltpu.sync_copy(outv, out_hbm.at[t])
            return carry
        lax.fori_loop(0, jmax, window, 0)

    return dec_kernel


def _tc_block(np_):
    blk = 1280
    grid = np_ // blk
    return blk, grid


def _make_tc1(np_):
    blk, grid = _tc_block(np_)

    def body(x_ref, w_ref, deg_ref, h_ref, hp_ref):
        s = lax.rsqrt(deg_ref[0] + deg_ref[1] + 1.0)
        h = jnp.dot(x_ref[...], w_ref[...], preferred_element_type=jnp.float32)
        h_ref[...] = h
        hp_ref[...] = h * s

    return pl.pallas_call(
        body,
        grid=(grid,),
        in_specs=[
            pl.BlockSpec((blk, D), lambda i: (i, 0)),
            pl.BlockSpec((D, D), lambda i: (0, 0)),
            pl.BlockSpec((NC, blk, D), lambda i: (0, i, 0)),
        ],
        out_specs=[
            pl.BlockSpec((blk, D), lambda i: (i, 0)),
            pl.BlockSpec((blk, D), lambda i: (i, 0)),
        ],
        out_shape=[
            jax.ShapeDtypeStruct((np_, D), jnp.float32),
            jax.ShapeDtypeStruct((np_, D), jnp.float32),
        ],
    )


def _make_tc2(np_):
    blk, grid = _tc_block(np_)

    def body(acc_ref, h1_ref, deg_ref, w2_ref, b1_ref, h2_ref, hp2_ref):
        s = lax.rsqrt(deg_ref[0] + deg_ref[1] + 1.0)
        a = acc_ref[0] + acc_ref[1]
        h1 = h1_ref[...]
        act = jnp.maximum(s * a - (s * s) * h1 + b1_ref[...], 0.0)
        h2 = jnp.dot(act, w2_ref[...], preferred_element_type=jnp.float32)
        h2_ref[...] = h2
        hp2_ref[...] = h2 * s

    return pl.pallas_call(
        body,
        grid=(grid,),
        in_specs=[
            pl.BlockSpec((NC, blk, D), lambda i: (0, i, 0)),
            pl.BlockSpec((blk, D), lambda i: (i, 0)),
            pl.BlockSpec((NC, blk, D), lambda i: (0, i, 0)),
            pl.BlockSpec((D, D), lambda i: (0, 0)),
            pl.BlockSpec((1, D), lambda i: (0, 0)),
        ],
        out_specs=[
            pl.BlockSpec((blk, D), lambda i: (i, 0)),
            pl.BlockSpec((blk, D), lambda i: (i, 0)),
        ],
        out_shape=[
            jax.ShapeDtypeStruct((np_, D), jnp.float32),
            jax.ShapeDtypeStruct((np_, D), jnp.float32),
        ],
    )


def _make_tc3(np_):
    blk, grid = _tc_block(np_)

    def body(acc_ref, h2_ref, deg_ref, b2_ref, z_ref):
        s = lax.rsqrt(deg_ref[0] + deg_ref[1] + 1.0)
        a = acc_ref[0] + acc_ref[1]
        z_ref[...] = s * a - (s * s) * h2_ref[...] + b2_ref[...]

    return pl.pallas_call(
        body,
        grid=(grid,),
        in_specs=[
            pl.BlockSpec((NC, blk, D), lambda i: (0, i, 0)),
            pl.BlockSpec((blk, D), lambda i: (i, 0)),
            pl.BlockSpec((NC, blk, D), lambda i: (0, i, 0)),
            pl.BlockSpec((1, D), lambda i: (0, 0)),
        ],
        out_specs=pl.BlockSpec((blk, D), lambda i: (i, 0)),
        out_shape=jax.ShapeDtypeStruct((np_, D), jnp.float32),
    )


def kernel(x, edge_index, edge_label_index, W1, b1, W2, b2):
    n, d = x.shape
    e = edge_index.shape[1]
    el = edge_label_index.shape[1]
    assert d == D
    np_ = ((n + 1279) // 1280) * 1280

    nch = -(-e // (NW * CH))
    nch += (-nch) % NB
    e_pad = NW * CH * nch
    if e_pad > e and np_ == n:
        np_ += 1280
    ei32 = edge_index.astype(jnp.int32)
    dummy = n + (jnp.arange(e_pad - e, dtype=jnp.int32) % (np_ - n))
    eip = jnp.concatenate(
        [ei32, jnp.stack([dummy, dummy])], axis=1)
    ei = eip.reshape(2, NW, nch, CH)
    pe = ((eip[0] << PBITS) | eip[1]).reshape(NW, nch, CH)
    eli = edge_label_index.astype(jnp.int32).reshape(2, el // DCH, 1, DCH)
    xp = jnp.pad(x, ((0, np_ - n), (0, 0)))
    b1r = b1.reshape(1, D)
    b2r = b2.reshape(1, D)

    degs = _make_deg(np_, nch)(ei)
    h1, hp1 = _make_tc1(np_)(xp, W1, degs)
    acc1 = _make_agg(np_, nch)(hp1, pe)
    h2, hp2 = _make_tc2(np_)(acc1, h1, degs, W2, b1r)
    acc2 = _make_agg(np_, nch)(hp2, pe)
    z = _make_tc3(np_)(acc2, h2, degs, b2r)
    out = _make_decode(np_, el)(z, eli)
    return out.reshape(el)
